# baseline (speedup 1.0000x reference)
import jax
import jax.numpy as jnp
from jax import lax
from jax.experimental import pallas as pl
from jax.experimental.pallas import tpu as pltpu
from jax.experimental.pallas import tpu_sc as plsc

_LAMADA = 0.0001
_FACTOR = 32
_BATCH = 16384
_NC = 2
_NS = 16
_NW = _NC * _NS
_BPW = _BATCH // _NW
_CHUNK = 128
_NCHUNK = _BPW // _CHUNK
_GROUPS = _BPW // 16


def _wbpr_body(user_hbm, item_hbm, intent_hbm, wu_hbm, wi_hbm, out_hbm,
               idx_u, idx_i, intent_v, u_rows, i_rows,
               part_v, red_v, out_v, shared, sem):
    cid = lax.axis_index("c")
    sid = lax.axis_index("s")
    wid = cid * _NS + sid

    pltpu.sync_copy(user_hbm.at[wid], idx_u)
    pltpu.sync_copy(item_hbm.at[wid], idx_i)
    pltpu.sync_copy(intent_hbm.at[wid], intent_v)

    copies = []
    for j in range(_NCHUNK):
        copies.append(pltpu.async_copy(
            wu_hbm.at[idx_u.at[j]], u_rows.at[pl.ds(j * _CHUNK, _CHUNK)], sem))
        copies.append(pltpu.async_copy(
            wi_hbm.at[idx_i.at[j]], i_rows.at[pl.ds(j * _CHUNK, _CHUNK)], sem))
    for c in copies:
        c.wait()

    zero = jnp.zeros((16,), jnp.float32)

    def group(g, carry):
        sse, squ, sqi = carry
        row0 = g * 16
        rows = row0 + lax.iota(jnp.int32, 16)
        pred = zero
        for d in range(_FACTOR):
            cols = jnp.full((16,), d, jnp.int32)
            uc = plsc.load_gather(u_rows, [rows, cols])
            ic = plsc.load_gather(i_rows, [rows, cols])
            pred = pred + uc * ic
            squ = squ + uc * uc
            sqi = sqi + ic * ic
        err = intent_v[pl.ds(row0, 16)] - pred
        return sse + err * err, squ, sqi

    sse, squ, sqi = lax.fori_loop(0, _GROUPS, group, (zero, zero, zero))

    part_v[0] = sse
    part_v[1] = squ
    part_v[2] = sqi
    for r in range(3, 16):
        part_v[r] = zero

    @pl.when(sid == 0)
    def _():
        pltpu.sync_copy(part_v, shared)

    plsc.subcore_barrier()

    @pl.when(sid != 0)
    def _():
        pltpu.sync_copy(part_v, shared.at[lax.iota(jnp.int32, 16)], add=True)

    plsc.subcore_barrier()

    @pl.when(sid == 0)
    def _():
        pltpu.sync_copy(shared, red_v)
        c_mse = jnp.float32(1.0 / _BATCH)
        c_l2 = jnp.float32(_LAMADA / (_BATCH * _FACTOR))
        vec = red_v[0] * c_mse + (red_v[1] + red_v[2]) * c_l2
        total = jnp.sum(vec)
        out_v[...] = jnp.full((16,), total, jnp.float32)
        pltpu.sync_copy(out_v, out_hbm.at[cid])


_wbpr_sc = pl.kernel(
    _wbpr_body,
    out_type=jax.ShapeDtypeStruct((_NC, 16), jnp.float32),
    mesh=plsc.VectorSubcoreMesh(core_axis_name="c", subcore_axis_name="s",
                                num_cores=_NC, num_subcores=_NS),
    compiler_params=pltpu.CompilerParams(needs_layout_passes=False,
                                         use_tc_tiling_on_sc=False),
    scratch_types=[
        pltpu.VMEM((_NCHUNK, _CHUNK), jnp.int32),
        pltpu.VMEM((_NCHUNK, _CHUNK), jnp.int32),
        pltpu.VMEM((_BPW,), jnp.float32),
        pltpu.VMEM((_BPW, _FACTOR), jnp.float32),
        pltpu.VMEM((_BPW, _FACTOR), jnp.float32),
        pltpu.VMEM((16, 16), jnp.float32),
        pltpu.VMEM((16, 16), jnp.float32),
        pltpu.VMEM((16,), jnp.float32),
        pltpu.VMEM_SHARED((16, 16), jnp.float32),
        pltpu.SemaphoreType.DMA,
    ],
    name="wbpr_sc",
)


def kernel(user, item_i, item_j, intention, W_user, W_item):
    del item_j
    user_r = user.astype(jnp.int32).reshape(_NW, _NCHUNK, _CHUNK)
    item_r = item_i.astype(jnp.int32).reshape(_NW, _NCHUNK, _CHUNK)
    intent_r = intention.astype(jnp.float32).reshape(_NW, _BPW)
    out = _wbpr_sc(user_r, item_r, intent_r, W_user, W_item)
    return out[0, 0] + out[1, 0]


# speedup vs baseline: 3.5468x; 3.5468x over previous
import jax
import jax.numpy as jnp
from jax import lax
from jax.experimental import pallas as pl
from jax.experimental.pallas import tpu as pltpu
from jax.experimental.pallas import tpu_sc as plsc

_LAMADA = 0.0001
_FACTOR = 32
_BATCH = 16384
_NC = 2
_NS = 16
_NW = _NC * _NS
_BPW = _BATCH // _NW
_CHUNK = 8
_NCHUNK = _BPW // _CHUNK


def _wbpr_body(user_hbm, item_hbm, intent_hbm, wtu_hbm, wti_hbm, out_hbm,
               idx_u, idx_i, intent_v, win_u, win_i,
               part_v, red_v, out_v, shared, sem):
    cid = lax.axis_index("c")
    sid = lax.axis_index("s")
    wid = cid * _NS + sid

    pltpu.sync_copy(user_hbm.at[wid], idx_u)
    pltpu.sync_copy(item_hbm.at[wid], idx_i)
    pltpu.sync_copy(intent_hbm.at[wid], intent_v)

    zero = jnp.zeros((16,), jnp.float32)
    half_iota = lax.iota(jnp.int32, 16)

    def chunk(c, carry):
        sse, squ, sqi = carry
        base = c * _CHUNK
        vu = idx_u[pl.ds(base, 16)]
        vi = idx_i[pl.ds(base, 16)]
        bu = vu >> 7
        bi = vi >> 7
        lu = vu & 127
        li = vi & 127
        tv = intent_v[pl.ds(base, 16)]
        copies = []
        for j in range(_CHUNK):
            cu = pl.multiple_of(bu[j] << 7, 128)
            ci = pl.multiple_of(bi[j] << 7, 128)
            copies.append(pltpu.async_copy(
                wtu_hbm.at[:, pl.ds(cu, 128)], win_u.at[j], sem))
            copies.append(pltpu.async_copy(
                wti_hbm.at[:, pl.ds(ci, 128)], win_i.at[j], sem))
        for cp in copies:
            cp.wait()

        for j in range(_CHUNK):
            ju = jnp.full((16,), lu[j], jnp.int32)
            ji = jnp.full((16,), li[j], jnp.int32)
            slot = jnp.full((16,), j, jnp.int32)
            u0 = plsc.load_gather(win_u, [slot, half_iota, ju])
            u1 = plsc.load_gather(win_u, [slot, 16 + half_iota, ju])
            i0 = plsc.load_gather(win_i, [slot, half_iota, ji])
            i1 = plsc.load_gather(win_i, [slot, 16 + half_iota, ji])
            squ = squ + u0 * u0 + u1 * u1
            sqi = sqi + i0 * i0 + i1 * i1
            pred = jnp.sum(u0 * i0 + u1 * i1)
            err = tv[j] - pred
            sse = sse + err * err
        return sse, squ, sqi

    sse_s, squ, sqi = lax.fori_loop(
        0, _NCHUNK, chunk, (jnp.float32(0.0), zero, zero))

    part_v[0] = jnp.full((16,), sse_s * 0.0625, jnp.float32)
    part_v[1] = squ
    part_v[2] = sqi
    for r in range(3, 16):
        part_v[r] = zero

    @pl.when(sid == 0)
    def _():
        pltpu.sync_copy(part_v, shared)

    plsc.subcore_barrier()

    @pl.when(sid != 0)
    def _():
        pltpu.sync_copy(part_v, shared.at[lax.iota(jnp.int32, 16)], add=True)

    plsc.subcore_barrier()

    @pl.when(sid == 0)
    def _():
        pltpu.sync_copy(shared, red_v)
        c_mse = jnp.float32(1.0 / _BATCH)
        c_l2 = jnp.float32(_LAMADA / (_BATCH * _FACTOR))
        vec = red_v[0] * c_mse + (red_v[1] + red_v[2]) * c_l2
        total = jnp.sum(vec)
        out_v[...] = jnp.full((16,), total, jnp.float32)
        pltpu.sync_copy(out_v, out_hbm.at[cid])


_wbpr_sc = pl.kernel(
    _wbpr_body,
    out_type=jax.ShapeDtypeStruct((_NC, 16), jnp.float32),
    mesh=plsc.VectorSubcoreMesh(core_axis_name="c", subcore_axis_name="s",
                                num_cores=_NC, num_subcores=_NS),
    compiler_params=pltpu.CompilerParams(needs_layout_passes=False),
    scratch_types=[
        pltpu.VMEM((_BPW,), jnp.int32),
        pltpu.VMEM((_BPW,), jnp.int32),
        pltpu.VMEM((_BPW,), jnp.float32),
        pltpu.VMEM((_CHUNK, _FACTOR, 128), jnp.float32),
        pltpu.VMEM((_CHUNK, _FACTOR, 128), jnp.float32),
        pltpu.VMEM((16, 16), jnp.float32),
        pltpu.VMEM((16, 16), jnp.float32),
        pltpu.VMEM((16,), jnp.float32),
        pltpu.VMEM_SHARED((16, 16), jnp.float32),
        pltpu.SemaphoreType.DMA,
    ],
    name="wbpr_sc",
)


def kernel(user, item_i, item_j, intention, W_user, W_item):
    del item_j
    user_r = user.astype(jnp.int32).reshape(_NW, _BPW)
    item_r = item_i.astype(jnp.int32).reshape(_NW, _BPW)
    intent_r = intention.astype(jnp.float32).reshape(_NW, _BPW)
    out = _wbpr_sc(user_r, item_r, intent_r, W_user.T, W_item.T)
    return out[0, 0] + out[1, 0]


# speedup vs baseline: 3.6249x; 1.0220x over previous
import jax
import jax.numpy as jnp
from jax import lax
from jax.experimental import pallas as pl
from jax.experimental.pallas import tpu as pltpu
from jax.experimental.pallas import tpu_sc as plsc

_LAMADA = 0.0001
_FACTOR = 32
_BATCH = 16384
_NC = 2
_NS = 16
_NW = _NC * _NS
_BPW = _BATCH // _NW
_CHUNK = 4
_NCHUNK = _BPW // _CHUNK
_IPAD = _BPW + 16


def _wbpr_body(user_hbm, item_hbm, intent_hbm, wtu_hbm, wti_hbm, out_hbm,
               idx_u, idx_i, intent_v, win_u, win_i,
               part_v, red_v, out_v, shared, sem_a, sem_b):
    cid = lax.axis_index("c")
    sid = lax.axis_index("s")
    wid = cid * _NS + sid

    pltpu.sync_copy(user_hbm.at[wid], idx_u.at[pl.ds(0, _BPW)])
    pltpu.sync_copy(item_hbm.at[wid], idx_i.at[pl.ds(0, _BPW)])
    pltpu.sync_copy(intent_hbm.at[wid], intent_v.at[pl.ds(0, _BPW)])

    zero = jnp.zeros((16,), jnp.float32)
    half_iota = lax.iota(jnp.int32, 16)

    def fire(c, p, sem):
        base = c * _CHUNK
        bu = idx_u[pl.ds(base, 16)] >> 7
        bi = idx_i[pl.ds(base, 16)] >> 7
        for j in range(_CHUNK):
            cu = pl.multiple_of(bu[j] << 7, 128)
            ci = pl.multiple_of(bi[j] << 7, 128)
            pltpu.async_copy(
                wtu_hbm.at[:, pl.ds(cu, 128)], win_u.at[p, j], sem)
            pltpu.async_copy(
                wti_hbm.at[:, pl.ds(ci, 128)], win_i.at[p, j], sem)

    def drain(p, sem):
        for j in range(_CHUNK):
            pltpu.make_async_copy(
                wtu_hbm.at[:, pl.ds(0, 128)], win_u.at[p, j], sem).wait()
            pltpu.make_async_copy(
                wti_hbm.at[:, pl.ds(0, 128)], win_i.at[p, j], sem).wait()

    def process(c, p, carry):
        sse, squ, sqi = carry
        base = c * _CHUNK
        lu = idx_u[pl.ds(base, 16)] & 127
        li = idx_i[pl.ds(base, 16)] & 127
        tv = intent_v[pl.ds(base, 16)]
        pvec = jnp.full((16,), p, jnp.int32)
        for j in range(_CHUNK):
            ju = jnp.full((16,), lu[j], jnp.int32)
            ji = jnp.full((16,), li[j], jnp.int32)
            slot = jnp.full((16,), j, jnp.int32)
            u0 = plsc.load_gather(win_u, [pvec, slot, half_iota, ju])
            u1 = plsc.load_gather(win_u, [pvec, slot, 16 + half_iota, ju])
            i0 = plsc.load_gather(win_i, [pvec, slot, half_iota, ji])
            i1 = plsc.load_gather(win_i, [pvec, slot, 16 + half_iota, ji])
            squ = squ + u0 * u0 + u1 * u1
            sqi = sqi + i0 * i0 + i1 * i1
            pred = jnp.sum(u0 * i0 + u1 * i1)
            err = tv[j] - pred
            sse = sse + err * err
        return sse, squ, sqi

    fire(0, 0, sem_a)

    def step(c, carry):
        p = c & 1

        @pl.when((c + 1 < _NCHUNK) & (p == 0))
        def _():
            fire(c + 1, 1, sem_b)

        @pl.when((c + 1 < _NCHUNK) & (p == 1))
        def _():
            fire(c + 1, 0, sem_a)

        def on_parity(p_static, sem):
            drain(p_static, sem)
            return process(c, p_static, carry)

        res0 = lax.cond(p == 0,
                        lambda: on_parity(0, sem_a),
                        lambda: on_parity(1, sem_b))
        return res0

    sse_s, squ, sqi = lax.fori_loop(
        0, _NCHUNK, step, (jnp.float32(0.0), zero, zero))

    part_v[0] = jnp.full((16,), sse_s * 0.0625, jnp.float32)
    part_v[1] = squ
    part_v[2] = sqi
    for r in range(3, 16):
        part_v[r] = zero

    @pl.when(sid == 0)
    def _():
        pltpu.sync_copy(part_v, shared)

    plsc.subcore_barrier()

    @pl.when(sid != 0)
    def _():
        pltpu.sync_copy(part_v, shared.at[lax.iota(jnp.int32, 16)], add=True)

    plsc.subcore_barrier()

    @pl.when(sid == 0)
    def _():
        pltpu.sync_copy(shared, red_v)
        c_mse = jnp.float32(1.0 / _BATCH)
        c_l2 = jnp.float32(_LAMADA / (_BATCH * _FACTOR))
        vec = red_v[0] * c_mse + (red_v[1] + red_v[2]) * c_l2
        total = jnp.sum(vec)
        out_v[...] = jnp.full((16,), total, jnp.float32)
        pltpu.sync_copy(out_v, out_hbm.at[cid])


_wbpr_sc = pl.kernel(
    _wbpr_body,
    out_type=jax.ShapeDtypeStruct((_NC, 16), jnp.float32),
    mesh=plsc.VectorSubcoreMesh(core_axis_name="c", subcore_axis_name="s",
                                num_cores=_NC, num_subcores=_NS),
    compiler_params=pltpu.CompilerParams(needs_layout_passes=False),
    scratch_types=[
        pltpu.VMEM((_IPAD,), jnp.int32),
        pltpu.VMEM((_IPAD,), jnp.int32),
        pltpu.VMEM((_IPAD,), jnp.float32),
        pltpu.VMEM((2, _CHUNK, _FACTOR, 128), jnp.float32),
        pltpu.VMEM((2, _CHUNK, _FACTOR, 128), jnp.float32),
        pltpu.VMEM((16, 16), jnp.float32),
        pltpu.VMEM((16, 16), jnp.float32),
        pltpu.VMEM((16,), jnp.float32),
        pltpu.VMEM_SHARED((16, 16), jnp.float32),
        pltpu.SemaphoreType.DMA,
        pltpu.SemaphoreType.DMA,
    ],
    name="wbpr_sc",
)


def kernel(user, item_i, item_j, intention, W_user, W_item):
    del item_j
    user_r = user.astype(jnp.int32).reshape(_NW, _BPW)
    item_r = item_i.astype(jnp.int32).reshape(_NW, _BPW)
    intent_r = intention.astype(jnp.float32).reshape(_NW, _BPW)
    out = _wbpr_sc(user_r, item_r, intent_r, W_user.T, W_item.T)
    return out[0, 0] + out[1, 0]


# speedup vs baseline: 3.9367x; 1.0860x over previous
import jax
import jax.numpy as jnp
from jax import lax
from jax.experimental import pallas as pl
from jax.experimental.pallas import tpu as pltpu
from jax.experimental.pallas import tpu_sc as plsc

_LAMADA = 0.0001
_FACTOR = 32
_BATCH = 16384
_NC = 2
_NS = 16
_NW = _NC * _NS
_BPW = _BATCH // _NW
_CHUNK = 4
_NCHUNK = _BPW // _CHUNK
_IPAD = _BPW + 16


def _wbpr_body(user_hbm, item_hbm, intent_hbm, wtu_hbm, wti_hbm, out_hbm,
               idx_u, idx_i, intent_v, win_u, win_i,
               part_v, red_v, out_v, shared, sem_a, sem_b, sem_c):
    cid = lax.axis_index("c")
    sid = lax.axis_index("s")
    wid = cid * _NS + sid

    pltpu.sync_copy(user_hbm.at[wid], idx_u.at[pl.ds(0, _BPW)])
    pltpu.sync_copy(item_hbm.at[wid], idx_i.at[pl.ds(0, _BPW)])
    pltpu.sync_copy(intent_hbm.at[wid], intent_v.at[pl.ds(0, _BPW)])

    zero = jnp.zeros((16,), jnp.float32)
    half_iota = lax.iota(jnp.int32, 16)

    def fire(c, p, sem):
        base = c * _CHUNK
        bu = idx_u[pl.ds(base, 16)] >> 7
        bi = idx_i[pl.ds(base, 16)] >> 7
        for j in range(_CHUNK):
            cu = pl.multiple_of(bu[j] << 7, 128)
            ci = pl.multiple_of(bi[j] << 7, 128)
            pltpu.async_copy(
                wtu_hbm.at[:, pl.ds(cu, 128)], win_u.at[p, j], sem)
            pltpu.async_copy(
                wti_hbm.at[:, pl.ds(ci, 128)], win_i.at[p, j], sem)

    def drain(p, sem):
        for j in range(_CHUNK):
            pltpu.make_async_copy(
                wtu_hbm.at[:, pl.ds(0, 128)], win_u.at[p, j], sem).wait()
            pltpu.make_async_copy(
                wti_hbm.at[:, pl.ds(0, 128)], win_i.at[p, j], sem).wait()

    def process(c, p, carry):
        sse, squ, sqi = carry
        base = c * _CHUNK
        lu = idx_u[pl.ds(base, 16)] & 127
        li = idx_i[pl.ds(base, 16)] & 127
        tv = intent_v[pl.ds(base, 16)]
        pvec = jnp.full((16,), p, jnp.int32)
        for j in range(_CHUNK):
            ju = jnp.full((16,), lu[j], jnp.int32)
            ji = jnp.full((16,), li[j], jnp.int32)
            slot = jnp.full((16,), j, jnp.int32)
            u0 = plsc.load_gather(win_u, [pvec, slot, half_iota, ju])
            u1 = plsc.load_gather(win_u, [pvec, slot, 16 + half_iota, ju])
            i0 = plsc.load_gather(win_i, [pvec, slot, half_iota, ji])
            i1 = plsc.load_gather(win_i, [pvec, slot, 16 + half_iota, ji])
            squ = squ + u0 * u0 + u1 * u1
            sqi = sqi + i0 * i0 + i1 * i1
            pred = jnp.sum(u0 * i0 + u1 * i1)
            err = tv[j] - pred
            sse = sse + err * err
        return sse, squ, sqi

    sems = (sem_a, sem_b, sem_c)
    fire(0, 0, sems[0])
    fire(1, 1, sems[1])

    def step(c, carry):
        nxt = c + 2

        for k in range(3):
            @pl.when((nxt < _NCHUNK) & (nxt % 3 == k))
            def _(k=k):
                fire(nxt, k, sems[k])

        def on_parity(p_static):
            drain(p_static, sems[p_static])
            return process(c, p_static, carry)

        return lax.cond(
            c % 3 == 0,
            lambda: on_parity(0),
            lambda: lax.cond(
                c % 3 == 1,
                lambda: on_parity(1),
                lambda: on_parity(2)))

    sse_s, squ, sqi = lax.fori_loop(
        0, _NCHUNK, step, (jnp.float32(0.0), zero, zero))

    part_v[0] = jnp.full((16,), sse_s * 0.0625, jnp.float32)
    part_v[1] = squ
    part_v[2] = sqi
    for r in range(3, 16):
        part_v[r] = zero

    @pl.when(sid == 0)
    def _():
        pltpu.sync_copy(part_v, shared)

    plsc.subcore_barrier()

    @pl.when(sid != 0)
    def _():
        pltpu.sync_copy(part_v, shared.at[lax.iota(jnp.int32, 16)], add=True)

    plsc.subcore_barrier()

    @pl.when(sid == 0)
    def _():
        pltpu.sync_copy(shared, red_v)
        c_mse = jnp.float32(1.0 / _BATCH)
        c_l2 = jnp.float32(_LAMADA / (_BATCH * _FACTOR))
        vec = red_v[0] * c_mse + (red_v[1] + red_v[2]) * c_l2
        total = jnp.sum(vec)
        out_v[...] = jnp.full((16,), total, jnp.float32)
        pltpu.sync_copy(out_v, out_hbm.at[cid])


_wbpr_sc = pl.kernel(
    _wbpr_body,
    out_type=jax.ShapeDtypeStruct((_NC, 16), jnp.float32),
    mesh=plsc.VectorSubcoreMesh(core_axis_name="c", subcore_axis_name="s",
                                num_cores=_NC, num_subcores=_NS),
    compiler_params=pltpu.CompilerParams(needs_layout_passes=False),
    scratch_types=[
        pltpu.VMEM((_IPAD,), jnp.int32),
        pltpu.VMEM((_IPAD,), jnp.int32),
        pltpu.VMEM((_IPAD,), jnp.float32),
        pltpu.VMEM((3, _CHUNK, _FACTOR, 128), jnp.float32),
        pltpu.VMEM((3, _CHUNK, _FACTOR, 128), jnp.float32),
        pltpu.VMEM((16, 16), jnp.float32),
        pltpu.VMEM((16, 16), jnp.float32),
        pltpu.VMEM((16,), jnp.float32),
        pltpu.VMEM_SHARED((16, 16), jnp.float32),
        pltpu.SemaphoreType.DMA,
        pltpu.SemaphoreType.DMA,
        pltpu.SemaphoreType.DMA,
    ],
    name="wbpr_sc",
)


def kernel(user, item_i, item_j, intention, W_user, W_item):
    del item_j
    user_r = user.astype(jnp.int32).reshape(_NW, _BPW)
    item_r = item_i.astype(jnp.int32).reshape(_NW, _BPW)
    intent_r = intention.astype(jnp.float32).reshape(_NW, _BPW)
    out = _wbpr_sc(user_r, item_r, intent_r, W_user.T, W_item.T)
    return out[0, 0] + out[1, 0]


# speedup vs baseline: 3.9547x; 1.0046x over previous
import jax
import jax.numpy as jnp
from jax import lax
from jax.experimental import pallas as pl
from jax.experimental.pallas import tpu as pltpu
from jax.experimental.pallas import tpu_sc as plsc

_LAMADA = 0.0001
_FACTOR = 32
_BATCH = 16384
_NC = 2
_NS = 16
_NW = _NC * _NS
_BPW = _BATCH // _NW
_CHUNK = 4
_NCHUNK = _BPW // _CHUNK
_IPAD = _BPW + 16


def _wbpr_body(user_hbm, item_hbm, intent_hbm, wtu_hbm, wti_hbm, out_hbm,
               idx_u, idx_i, intent_v, win_u, win_i,
               part_v, red_v, out_v, shared, sem_a, sem_b, sem_c):
    cid = lax.axis_index("c")
    sid = lax.axis_index("s")
    wid = cid * _NS + sid

    pltpu.sync_copy(user_hbm.at[wid], idx_u.at[pl.ds(0, _BPW)])
    pltpu.sync_copy(item_hbm.at[wid], idx_i.at[pl.ds(0, _BPW)])
    pltpu.sync_copy(intent_hbm.at[wid], intent_v.at[pl.ds(0, _BPW)])

    zero = jnp.zeros((16,), jnp.float32)
    half_iota = lax.iota(jnp.int32, 16)

    def fire(c, p, sem):
        base = c * _CHUNK
        bu = idx_u[pl.ds(base, 16)] >> 7
        bi = idx_i[pl.ds(base, 16)] >> 7
        for j in range(_CHUNK):
            cu = pl.multiple_of(bu[j] << 7, 128)
            ci = pl.multiple_of(bi[j] << 7, 128)
            for b in range(4):
                pltpu.async_copy(
                    wtu_hbm.at[pl.ds(8 * b, 8), pl.ds(cu, 128)],
                    win_u.at[p, j, pl.ds(8 * b, 8)], sem)
                pltpu.async_copy(
                    wti_hbm.at[pl.ds(8 * b, 8), pl.ds(ci, 128)],
                    win_i.at[p, j, pl.ds(8 * b, 8)], sem)

    def drain(p, sem):
        for j in range(_CHUNK):
            pltpu.make_async_copy(
                wtu_hbm.at[:, pl.ds(0, 128)], win_u.at[p, j], sem).wait()
            pltpu.make_async_copy(
                wti_hbm.at[:, pl.ds(0, 128)], win_i.at[p, j], sem).wait()

    def process(c, p, carry):
        sse, squ, sqi = carry
        base = c * _CHUNK
        lu = idx_u[pl.ds(base, 16)] & 127
        li = idx_i[pl.ds(base, 16)] & 127
        tv = intent_v[pl.ds(base, 16)]
        pvec = jnp.full((16,), p, jnp.int32)
        for j in range(_CHUNK):
            ju = jnp.full((16,), lu[j], jnp.int32)
            ji = jnp.full((16,), li[j], jnp.int32)
            slot = jnp.full((16,), j, jnp.int32)
            u0 = plsc.load_gather(win_u, [pvec, slot, half_iota, ju])
            u1 = plsc.load_gather(win_u, [pvec, slot, 16 + half_iota, ju])
            i0 = plsc.load_gather(win_i, [pvec, slot, half_iota, ji])
            i1 = plsc.load_gather(win_i, [pvec, slot, 16 + half_iota, ji])
            squ = squ + u0 * u0 + u1 * u1
            sqi = sqi + i0 * i0 + i1 * i1
            pred = jnp.sum(u0 * i0 + u1 * i1)
            err = tv[j] - pred
            sse = sse + err * err
        return sse, squ, sqi

    sems = (sem_a, sem_b, sem_c)
    fire(0, 0, sems[0])
    fire(1, 1, sems[1])

    def step(c, carry):
        nxt = c + 2

        for k in range(3):
            @pl.when((nxt < _NCHUNK) & (nxt % 3 == k))
            def _(k=k):
                fire(nxt, k, sems[k])

        def on_parity(p_static):
            drain(p_static, sems[p_static])
            return process(c, p_static, carry)

        return lax.cond(
            c % 3 == 0,
            lambda: on_parity(0),
            lambda: lax.cond(
                c % 3 == 1,
                lambda: on_parity(1),
                lambda: on_parity(2)))

    sse_s, squ, sqi = lax.fori_loop(
        0, _NCHUNK, step, (jnp.float32(0.0), zero, zero))

    part_v[0] = jnp.full((16,), sse_s * 0.0625, jnp.float32)
    part_v[1] = squ
    part_v[2] = sqi
    for r in range(3, 16):
        part_v[r] = zero

    @pl.when(sid == 0)
    def _():
        pltpu.sync_copy(part_v, shared)

    plsc.subcore_barrier()

    @pl.when(sid != 0)
    def _():
        pltpu.sync_copy(part_v, shared.at[lax.iota(jnp.int32, 16)], add=True)

    plsc.subcore_barrier()

    @pl.when(sid == 0)
    def _():
        pltpu.sync_copy(shared, red_v)
        c_mse = jnp.float32(1.0 / _BATCH)
        c_l2 = jnp.float32(_LAMADA / (_BATCH * _FACTOR))
        vec = red_v[0] * c_mse + (red_v[1] + red_v[2]) * c_l2
        total = jnp.sum(vec)
        out_v[...] = jnp.full((16,), total, jnp.float32)
        pltpu.sync_copy(out_v, out_hbm.at[cid])


_wbpr_sc = pl.kernel(
    _wbpr_body,
    out_type=jax.ShapeDtypeStruct((_NC, 16), jnp.float32),
    mesh=plsc.VectorSubcoreMesh(core_axis_name="c", subcore_axis_name="s",
                                num_cores=_NC, num_subcores=_NS),
    compiler_params=pltpu.CompilerParams(needs_layout_passes=False),
    scratch_types=[
        pltpu.VMEM((_IPAD,), jnp.int32),
        pltpu.VMEM((_IPAD,), jnp.int32),
        pltpu.VMEM((_IPAD,), jnp.float32),
        pltpu.VMEM((3, _CHUNK, _FACTOR, 128), jnp.float32),
        pltpu.VMEM((3, _CHUNK, _FACTOR, 128), jnp.float32),
        pltpu.VMEM((16, 16), jnp.float32),
        pltpu.VMEM((16, 16), jnp.float32),
        pltpu.VMEM((16,), jnp.float32),
        pltpu.VMEM_SHARED((16, 16), jnp.float32),
        pltpu.SemaphoreType.DMA,
        pltpu.SemaphoreType.DMA,
        pltpu.SemaphoreType.DMA,
    ],
    name="wbpr_sc",
)


def kernel(user, item_i, item_j, intention, W_user, W_item):
    del item_j
    user_r = user.astype(jnp.int32).reshape(_NW, _BPW)
    item_r = item_i.astype(jnp.int32).reshape(_NW, _BPW)
    intent_r = intention.astype(jnp.float32).reshape(_NW, _BPW)
    out = _wbpr_sc(user_r, item_r, intent_r, W_user.T, W_item.T)
    return out[0, 0] + out[1, 0]
